# baseline (speedup 1.0000x reference)
import functools

import jax
import jax.numpy as jnp
from jax import lax
from jax.experimental import pallas as pl
from jax.experimental.pallas import tpu as pltpu
from jax.experimental.pallas import tpu_sc as plsc

B = 16384
SEQ = 50
D = 32
NW = 32
RPW = B // NW
CB = 16
NCH = RPW // CB
G = 100
SPC = CB * SEQ // G
NB = 4

_mesh = plsc.VectorSubcoreMesh(core_axis_name="c", subcore_axis_name="s")


@functools.partial(
    pl.kernel,
    mesh=_mesh,
    out_type=jax.ShapeDtypeStruct((B, D), jnp.float32),
    scratch_types=[
        pltpu.VMEM((NB, SPC, G), jnp.int32),
        pltpu.VMEM((SPC, G, D), jnp.float32),
        pltpu.VMEM((SPC, G, D), jnp.float32),
        pltpu.VMEM((SPC, G, D), jnp.float32),
        pltpu.VMEM((SPC, G, D), jnp.float32),
        pltpu.VMEM((CB, D), jnp.float32),
        pltpu.SemaphoreType.DMA,
        pltpu.SemaphoreType.DMA,
        pltpu.SemaphoreType.DMA,
        pltpu.SemaphoreType.DMA,
    ],
    compiler_params=pltpu.CompilerParams(use_tc_tiling_on_sc=False),
)
def _sum_embed(idx_hbm, t_hbm, out_hbm, gidx_v, rows0_v, rows1_v, rows2_v,
               rows3_v, out_v, sem0, sem1, sem2, sem3):
    ci = lax.axis_index("c")
    si = lax.axis_index("s")
    wid = si * 2 + ci
    rbase = wid * RPW

    rows_bufs = (rows0_v, rows1_v, rows2_v, rows3_v)
    sems = (sem0, sem1, sem2, sem3)

    def fire(c, par):
        pltpu.sync_copy(idx_hbm.at[pl.ds((rbase + c * CB) * SEQ // G, SPC)],
                        gidx_v.at[par])

        def g_body(g, carry):
            pltpu.async_copy(
                t_hbm.at[gidx_v.at[par, g]],
                rows_bufs[par].at[g],
                sems[par],
            )
            return carry

        lax.fori_loop(0, SPC, g_body, 0)

    def drain(par):
        def w_body(g, carry):
            pltpu.make_async_copy(
                t_hbm.at[pl.ds(0, G)],
                rows_bufs[par].at[g],
                sems[par],
            ).wait()
            return carry

        lax.fori_loop(0, SPC, w_body, 0)

    def accumulate(c, par):
        rows_v = rows_bufs[par]

        def row_body(r, carry):
            g = r // 2
            p = (r % 2) * SEQ
            a0 = rows_v[g, p + 0, pl.ds(0, 16)]
            a1 = rows_v[g, p + 0, pl.ds(16, 16)]
            b0 = rows_v[g, p + 1, pl.ds(0, 16)]
            b1 = rows_v[g, p + 1, pl.ds(16, 16)]
            for j in range(2, SEQ, 2):
                a0 = a0 + rows_v[g, p + j, pl.ds(0, 16)]
                a1 = a1 + rows_v[g, p + j, pl.ds(16, 16)]
                b0 = b0 + rows_v[g, p + j + 1, pl.ds(0, 16)]
                b1 = b1 + rows_v[g, p + j + 1, pl.ds(16, 16)]
            out_v[r, pl.ds(0, 16)] = a0 + b0
            out_v[r, pl.ds(16, 16)] = a1 + b1
            return carry

        lax.fori_loop(0, CB, row_body, 0)
        pltpu.sync_copy(out_v, out_hbm.at[pl.ds(rbase + c * CB, CB)])

    for b in range(NB - 1):
        fire(b, b)

    def ring_body(p, carry):
        c0 = NB * p
        for b in range(NB):
            c = c0 + b

            @pl.when(c + NB - 1 < NCH)
            def _():
                fire(c + NB - 1, (b + NB - 1) % NB)

            drain(b)
            accumulate(c, b)
        return carry

    lax.fori_loop(0, NCH // NB, ring_body, 0)


def kernel(input, table):
    idx = input.astype(jnp.int32).reshape(B * SEQ // G, G)
    return _sum_embed(idx, table)
